# baseline (speedup 1.0000x reference)
import functools

import jax
import jax.numpy as jnp
from jax import lax
from jax.experimental import pallas as pl
from jax.experimental.pallas import tpu as pltpu
from jax.experimental.pallas import tpu_sc as plsc

N = 10000
D = 128
N_PAD = 10240
NW = 32
CHUNK = 128
RPT = N_PAD // 16
HALF = D // 2
PHASES = 2

@functools.lru_cache(maxsize=None)
def _mesh():
    return plsc.VectorSubcoreMesh(core_axis_name="c", subcore_axis_name="s")


@functools.lru_cache(maxsize=None)
def _deg_kernel(chunks: int):
    @functools.partial(
        pl.kernel,
        out_type=jax.ShapeDtypeStruct((2, N_PAD), jnp.float32),
        mesh=_mesh(),
        scratch_types=[
            pltpu.VMEM_SHARED((N_PAD,), jnp.float32),
            pltpu.VMEM((chunks, CHUNK), jnp.int32),
            pltpu.VMEM((CHUNK,), jnp.float32),
        ],
    )
    def deg(dst_hbm, ones_hbm, zeros_hbm, out_hbm, acc_sh, idx_v, ones_v):
        c = lax.axis_index("c")
        s = lax.axis_index("s")
        wid = s * 2 + c
        pltpu.sync_copy(dst_hbm.at[wid], idx_v)
        pltpu.sync_copy(ones_hbm, ones_v)
        pltpu.sync_copy(zeros_hbm, acc_sh.at[pl.ds(s * RPT, RPT)])
        plsc.subcore_barrier()

        def body(j, carry):
            pltpu.sync_copy(ones_v, acc_sh.at[idx_v.at[j]], add=True)
            return carry

        lax.fori_loop(0, chunks, body, 0)
        plsc.subcore_barrier()
        pltpu.sync_copy(acc_sh.at[pl.ds(s * RPT, RPT)],
                        out_hbm.at[c, pl.ds(s * RPT, RPT)])

    return deg


@functools.lru_cache(maxsize=None)
def _agg_kernel(chunks: int):
    @functools.partial(
        pl.kernel,
        out_type=jax.ShapeDtypeStruct((2, 2, N_PAD, HALF), jnp.float32),
        mesh=_mesh(),
        scratch_types=[
            pltpu.VMEM_SHARED((N_PAD, HALF), jnp.float32),
            pltpu.VMEM_SHARED((N_PAD, HALF), jnp.float32),
            pltpu.VMEM((chunks, CHUNK), jnp.int32),
            pltpu.VMEM((chunks, CHUNK), jnp.int32),
            pltpu.VMEM((CHUNK, HALF), jnp.float32),
        ],
    )
    def agg(g_hbm, src_hbm, dst_hbm, zeros_hbm, out_hbm,
            g_sh, acc_sh, src_v, dst_v, rows_v):
        c = lax.axis_index("c")
        s = lax.axis_index("s")
        wid = s * 2 + c
        pltpu.sync_copy(src_hbm.at[wid], src_v)
        pltpu.sync_copy(dst_hbm.at[wid], dst_v)

        for k in range(2):
            pltpu.sync_copy(g_hbm.at[k, pl.ds(s * RPT, RPT)],
                            g_sh.at[pl.ds(s * RPT, RPT)])
            pltpu.sync_copy(zeros_hbm, acc_sh.at[pl.ds(s * RPT, RPT)])
            plsc.subcore_barrier()

            def body(j, carry):
                pltpu.sync_copy(g_sh.at[src_v.at[j]], rows_v)
                pltpu.sync_copy(rows_v, acc_sh.at[dst_v.at[j]], add=True)
                return carry

            lax.fori_loop(0, chunks, body, 0)
            plsc.subcore_barrier()
            pltpu.sync_copy(acc_sh.at[pl.ds(s * RPT, RPT)],
                            out_hbm.at[c, k, pl.ds(s * RPT, RPT)])

    return agg


def _dinv_body(deg_ref, dinv_ref):
    d = deg_ref[0:1, :] + deg_ref[1:2, :] + 1.0
    n = lax.broadcasted_iota(jnp.int32, (1, N_PAD), 1)
    ok = (n < N) & (d > 0)
    dinv_ref[...] = jnp.where(ok, lax.rsqrt(jnp.maximum(d, 1e-12)), 0.0)


def _scale_mm_body(x_ref, w_ref, dinv_ref, g_ref):
    g = jnp.dot(x_ref[...], w_ref[...],
                preferred_element_type=jnp.float32) * dinv_ref[...]
    g_ref[0] = g[:, :HALF]
    g_ref[1] = g[:, HALF:]


def _mid_body(s_ref, g_ref, dinv_ref, b_ref, w_ref, g2_ref):
    t = jnp.concatenate(
        [s_ref[0, 0] + s_ref[1, 0] + g_ref[0],
         s_ref[0, 1] + s_ref[1, 1] + g_ref[1]], axis=1)
    h = jnp.maximum(dinv_ref[...] * t + b_ref[...], 0.0)
    g2 = jnp.dot(h, w_ref[...],
                 preferred_element_type=jnp.float32) * dinv_ref[...]
    g2_ref[0] = g2[:, :HALF]
    g2_ref[1] = g2[:, HALF:]


def _fin_body(s_ref, g_ref, dinv_ref, b_ref, wo_ref, bo_ref, p_ref):
    t = jnp.concatenate(
        [s_ref[0, 0] + s_ref[1, 0] + g_ref[0],
         s_ref[0, 1] + s_ref[1, 1] + g_ref[1]], axis=1)
    h = jnp.maximum(dinv_ref[...] * t + b_ref[...], 0.0)
    logit = jnp.sum(h * wo_ref[...], axis=1, keepdims=True) + bo_ref[0, 0]
    n = lax.broadcasted_iota(jnp.int32, (N_PAD, 1), 0)
    mask = n < N
    logit = jnp.where(mask, logit, -jnp.inf)
    m = jnp.max(logit)
    e = jnp.where(mask, jnp.exp(logit - m), 0.0)
    p_ref[...] = e / jnp.sum(e)


def kernel(x, edge_index, W1, b1, W2, b2, Wo, bo):
    E = edge_index.shape[1]
    q = 2 * PHASES * CHUNK
    epw = q * ((E + NW * q - 1) // (NW * q))
    chunks = epw // CHUNK
    pad = epw * NW - E
    padv = jnp.full((pad,), N, jnp.int32)
    src = jnp.concatenate([edge_index[0], padv]).reshape(NW, chunks, CHUNK)
    dst = jnp.concatenate([edge_index[1], padv]).reshape(NW, chunks, CHUNK)
    x_pad = jnp.concatenate(
        [x.astype(jnp.float32), jnp.zeros((N_PAD - N, D), jnp.float32)])
    zeros_rows = jnp.zeros((RPT, HALF), jnp.float32)
    zeros_deg = jnp.zeros((RPT,), jnp.float32)
    ones_chunk = jnp.ones((CHUNK,), jnp.float32)

    deg2 = _deg_kernel(chunks)(dst, ones_chunk, zeros_deg)

    dinv_row = pl.pallas_call(
        _dinv_body,
        out_shape=jax.ShapeDtypeStruct((1, N_PAD), jnp.float32),
    )(deg2)
    dinv_col = dinv_row.reshape(N_PAD, 1)

    g1 = pl.pallas_call(
        _scale_mm_body,
        out_shape=jax.ShapeDtypeStruct((2, N_PAD, HALF), jnp.float32),
    )(x_pad, W1, dinv_col)

    S1 = _agg_kernel(chunks)(g1, src, dst, zeros_rows)

    g2 = pl.pallas_call(
        _mid_body,
        out_shape=jax.ShapeDtypeStruct((2, N_PAD, HALF), jnp.float32),
    )(S1, g1, dinv_col, b1.reshape(1, D), W2)

    S2 = _agg_kernel(chunks)(g2, src, dst, zeros_rows)

    p = pl.pallas_call(
        _fin_body,
        out_shape=jax.ShapeDtypeStruct((N_PAD, 1), jnp.float32),
    )(S2, g2, dinv_col, b2.reshape(1, D), Wo.reshape(1, D), bo.reshape(1, 1))

    return p[:N, 0]
